# Initial kernel scaffold; baseline (speedup 1.0000x reference)
import jax
import jax.numpy as jnp
from jax.experimental import pallas as pl


def kernel(users, items, adj_rows, adj_cols, adj_vals, user_emb, item_emb, W_gc_0, b_gc_0, W_bi_0, b_bi_0, W_gc_1, b_gc_1, W_bi_1, b_bi_1, W_gc_2, b_gc_2, W_bi_2, b_bi_2):
    raise NotImplementedError("write your pallas kernel here")


# speedup vs baseline: 3.2109x; 3.2109x over previous
import functools

import jax
import jax.numpy as jnp
from jax import lax
from jax.experimental import pallas as pl
from jax.experimental.pallas import tpu as pltpu
from jax.experimental.pallas import tpu_sc as plsc

NUSERS = 25000
NN = 50000
NP = 51200
D = 64
HD = 32
NNZ = 800000
BATCH = 4096
NC, NS = 2, 16

CH = 128
CPS = 4
SUPS = 98
EDGES_SUB = SUPS * CPS * CH
NNZ_PAD = EDGES_SUB * NS

RPS = NP // NS
ZROWS = 640
ZK = RPS // ZROWS

_SC_MESH = plsc.VectorSubcoreMesh(core_axis_name="c", subcore_axis_name="s")


@functools.partial(
    pl.kernel,
    out_type=jax.ShapeDtypeStruct((2 * NP, HD), jnp.float32),
    mesh=_SC_MESH,
    scratch_types=[
        pltpu.VMEM_SHARED((NP, HD), jnp.float32),
        pltpu.VMEM((CPS, CH), jnp.int32),
        pltpu.VMEM((CPS * CH, 16), jnp.float32),
        pltpu.VMEM((CPS, CH), jnp.int32),
        pltpu.VMEM((CPS * CH, HD), jnp.float32),
        pltpu.SemaphoreType.DMA,
    ],
    compiler_params=pltpu.CompilerParams(use_tc_tiling_on_sc=False),
)
def _spmm(ego_h, col_h, val_h, row_h, zero_h, out_h,
          side, colblk, valblk, rowblk, gath, sem):
    c = lax.axis_index("c")
    s = lax.axis_index("s")

    for k in range(ZK):
        pltpu.sync_copy(zero_h, side.at[pl.ds(s * RPS + k * ZROWS, ZROWS)])
    plsc.subcore_barrier()

    def _sup(u, _):
        pltpu.sync_copy(col_h.at[c, s, u], colblk)
        pltpu.sync_copy(val_h.at[s, u], valblk)
        pltpu.sync_copy(row_h.at[s, u], rowblk)
        cps = [
            pltpu.async_copy(ego_h.at[colblk.at[j]],
                             gath.at[pl.ds(j * CH, CH)], sem)
            for j in range(CPS)
        ]
        for cp in cps:
            cp.wait()
        def _scale(e, _):
            v = valblk[e, pl.ds(0, 16)]
            a = gath[e, pl.ds(0, 16)]
            b = gath[e, pl.ds(16, 16)]
            gath[e, pl.ds(0, 16)] = a * v
            gath[e, pl.ds(16, 16)] = b * v
            return 0
        lax.fori_loop(0, CPS * CH, _scale, 0)
        for j in range(CPS):
            pltpu.sync_copy(gath.at[pl.ds(j * CH, CH)],
                            side.at[rowblk.at[j]], add=True)
        return 0
    lax.fori_loop(0, SUPS, _sup, 0)
    plsc.subcore_barrier()

    for k in range(ZK):
        start = s * RPS + k * ZROWS
        pltpu.sync_copy(side.at[pl.ds(start, ZROWS)],
                        out_h.at[pl.ds(c * NP + start, ZROWS)])


BN = 1600
NBLK = NP // BN


def _dense_body(slo, shi, elo, ehi, wgc, bgc, wbi, bbi, egon, nrm):
    slo_, shi_ = slo[0], shi[0]
    elo_, ehi_ = elo[0], ehi[0]
    wgc_, wbi_ = wgc[...], wbi[...]
    x = (jnp.dot(slo_, wgc_[:HD], preferred_element_type=jnp.float32)
         + jnp.dot(shi_, wgc_[HD:], preferred_element_type=jnp.float32)
         + bgc[...])
    x = jnp.maximum(x, 0.2 * x)
    y = (jnp.dot(elo_ * slo_, wbi_[:HD], preferred_element_type=jnp.float32)
         + jnp.dot(ehi_ * shi_, wbi_[HD:], preferred_element_type=jnp.float32)
         + bbi[...])
    y = jnp.maximum(y, 0.2 * y)
    e = x + y
    n = jnp.sqrt(jnp.sum(e * e, axis=1, keepdims=True))
    nrm[...] = e / jnp.maximum(n, 1e-12)
    egon[0] = e[:, :HD]
    egon[1] = e[:, HD:]


_dense = pl.pallas_call(
    _dense_body,
    grid=(NBLK,),
    in_specs=[
        pl.BlockSpec((1, BN, HD), lambda i: (0, i, 0)),
        pl.BlockSpec((1, BN, HD), lambda i: (1, i, 0)),
        pl.BlockSpec((1, BN, HD), lambda i: (0, i, 0)),
        pl.BlockSpec((1, BN, HD), lambda i: (1, i, 0)),
        pl.BlockSpec((D, D), lambda i: (0, 0)),
        pl.BlockSpec((1, D), lambda i: (0, 0)),
        pl.BlockSpec((D, D), lambda i: (0, 0)),
        pl.BlockSpec((1, D), lambda i: (0, 0)),
    ],
    out_specs=[
        pl.BlockSpec((2, BN, HD), lambda i: (0, i, 0)),
        pl.BlockSpec((BN, D), lambda i: (i, 0)),
    ],
    out_shape=[
        jax.ShapeDtypeStruct((2, NP, HD), jnp.float32),
        jax.ShapeDtypeStruct((NP, D), jnp.float32),
    ],
)


GB = BATCH // (NC * NS)


@functools.partial(
    pl.kernel,
    out_type=(jax.ShapeDtypeStruct((4, BATCH, D), jnp.float32),
              jax.ShapeDtypeStruct((4, BATCH, D), jnp.float32)),
    mesh=_SC_MESH,
    scratch_types=[
        pltpu.VMEM((GB,), jnp.int32),
        pltpu.VMEM((GB, D), jnp.float32),
        pltpu.SemaphoreType.DMA,
    ],
    compiler_params=pltpu.CompilerParams(use_tc_tiling_on_sc=False),
)
def _gather(t0, t1, t2, t3, uidx_h, iidx_h, uout, iout, idxv, rows, sem):
    c = lax.axis_index("c")
    s = lax.axis_index("s")
    wid = s * NC + c
    base = wid * GB
    tables = (t0, t1, t2, t3)
    pltpu.sync_copy(uidx_h.at[pl.ds(base, GB)], idxv)
    for t in range(4):
        pltpu.async_copy(tables[t].at[idxv], rows, sem).wait()
        pltpu.sync_copy(rows, uout.at[t, pl.ds(base, GB)])
    pltpu.sync_copy(iidx_h.at[pl.ds(base, GB)], idxv)
    for t in range(4):
        pltpu.async_copy(tables[t].at[idxv], rows, sem).wait()
        pltpu.sync_copy(rows, iout.at[t, pl.ds(base, GB)])


DB = 512


def _dot_body(u, v, o):
    o[...] = jnp.sum(u[...] * v[...], axis=(0, 2)).reshape(1, 1, DB)


_dot = pl.pallas_call(
    _dot_body,
    grid=(BATCH // DB,),
    in_specs=[
        pl.BlockSpec((4, DB, D), lambda i: (0, i, 0)),
        pl.BlockSpec((4, DB, D), lambda i: (0, i, 0)),
    ],
    out_specs=pl.BlockSpec((1, 1, DB), lambda i: (i, 0, 0)),
    out_shape=jax.ShapeDtypeStruct((BATCH // DB, 1, DB), jnp.float32),
)


def kernel(users, items, adj_rows, adj_cols, adj_vals, user_emb, item_emb,
           W_gc_0, b_gc_0, W_bi_0, b_bi_0,
           W_gc_1, b_gc_1, W_bi_1, b_bi_1,
           W_gc_2, b_gc_2, W_bi_2, b_bi_2):
    ego0 = jnp.concatenate([user_emb, item_emb], axis=0)
    zrow = jnp.zeros((NP - NN, HD), jnp.float32)
    ego2 = jnp.concatenate([user_emb[:, :HD], item_emb[:, :HD], zrow,
                            user_emb[:, HD:], item_emb[:, HD:], zrow], axis=0)
    pad = NNZ_PAD - NNZ
    colp = jnp.concatenate([adj_cols.astype(jnp.int32),
                            jnp.zeros((pad,), jnp.int32)])
    rowp = jnp.concatenate([adj_rows.astype(jnp.int32),
                            jnp.zeros((pad,), jnp.int32)])
    valp = jnp.concatenate([adj_vals, jnp.zeros((pad,), jnp.float32)])
    col2 = jnp.stack([colp, colp + NP]).reshape(2, NS, SUPS, CPS, CH)
    rowr = rowp.reshape(NS, SUPS, CPS, CH)
    valr = jnp.broadcast_to(valp[:, None],
                            (NNZ_PAD, 16)).reshape(NS, SUPS, CPS * CH, 16)
    Ws = [(W_gc_0, b_gc_0, W_bi_0, b_bi_0),
          (W_gc_1, b_gc_1, W_bi_1, b_bi_1),
          (W_gc_2, b_gc_2, W_bi_2, b_bi_2)]
    zero_blk = jnp.zeros((ZROWS, HD), jnp.float32)
    norms = []
    for k in range(3):
        side2 = _spmm(ego2, col2, valr, rowr, zero_blk)
        egon, nrm = _dense(side2.reshape(2, NP, HD), side2.reshape(2, NP, HD),
                           ego2.reshape(2, NP, HD), ego2.reshape(2, NP, HD),
                           *Ws[k])
        ego2 = egon.reshape(2 * NP, HD)
        norms.append(nrm)
    uidx = users.astype(jnp.int32)
    iidx = items.astype(jnp.int32) + NUSERS
    ug, ig = _gather(ego0, norms[0], norms[1], norms[2], uidx, iidx)
    return _dot(ug, ig).reshape(BATCH)


# speedup vs baseline: 4.0191x; 1.2517x over previous
import functools

import jax
import jax.numpy as jnp
from jax import lax
from jax.experimental import pallas as pl
from jax.experimental.pallas import tpu as pltpu
from jax.experimental.pallas import tpu_sc as plsc

NUSERS = 25000
NN = 50000
NP = 51200
D = 64
HD = 32
NNZ = 800000
BATCH = 4096
NC, NS = 2, 16

CH = 128
CPS = 4
SUPS = 98
EDGES_SUB = SUPS * CPS * CH
NNZ_PAD = EDGES_SUB * NS

RPS = NP // NS
ZROWS = 640
ZK = RPS // ZROWS

_SC_MESH = plsc.VectorSubcoreMesh(core_axis_name="c", subcore_axis_name="s")


@functools.partial(
    pl.kernel,
    out_type=jax.ShapeDtypeStruct((2 * NP, HD), jnp.float32),
    mesh=_SC_MESH,
    scratch_types=[
        pltpu.VMEM_SHARED((NP, HD), jnp.float32),
        pltpu.VMEM((CPS, CH), jnp.int32),
        pltpu.VMEM((CPS * CH, 16), jnp.float32),
        pltpu.VMEM((CPS, CH), jnp.int32),
        pltpu.VMEM((CPS * CH, HD), jnp.float32),
        pltpu.SemaphoreType.DMA,
    ],
    compiler_params=pltpu.CompilerParams(use_tc_tiling_on_sc=False),
)
def _spmm(ego_h, col_h, val_h, row_h, zero_h, out_h,
          side, colblk, valblk, rowblk, gath, sem):
    c = lax.axis_index("c")
    s = lax.axis_index("s")

    for k in range(ZK):
        pltpu.sync_copy(zero_h, side.at[pl.ds(s * RPS + k * ZROWS, ZROWS)])
    plsc.subcore_barrier()

    def _sup(u, _):
        pltpu.sync_copy(col_h.at[c, s, u], colblk)
        pltpu.sync_copy(val_h.at[s, u], valblk)
        pltpu.sync_copy(row_h.at[s, u], rowblk)
        cps = [
            pltpu.async_copy(ego_h.at[colblk.at[j]],
                             gath.at[pl.ds(j * CH, CH)], sem)
            for j in range(CPS)
        ]
        for cp in cps:
            cp.wait()
        @plsc.parallel_loop(0, CPS * CH, unroll=8)
        def _scale(e):
            v = valblk[e, pl.ds(0, 16)]
            a = gath[e, pl.ds(0, 16)]
            b = gath[e, pl.ds(16, 16)]
            gath[e, pl.ds(0, 16)] = a * v
            gath[e, pl.ds(16, 16)] = b * v
        for j in range(CPS):
            pltpu.sync_copy(gath.at[pl.ds(j * CH, CH)],
                            side.at[rowblk.at[j]], add=True)
        return 0
    lax.fori_loop(0, SUPS, _sup, 0)
    plsc.subcore_barrier()

    for k in range(ZK):
        start = s * RPS + k * ZROWS
        pltpu.sync_copy(side.at[pl.ds(start, ZROWS)],
                        out_h.at[pl.ds(c * NP + start, ZROWS)])


BN = 1600
NBLK = NP // BN


def _dense_body(slo, shi, elo, ehi, wgc, bgc, wbi, bbi, egon, nrm):
    slo_, shi_ = slo[0], shi[0]
    elo_, ehi_ = elo[0], ehi[0]
    wgc_, wbi_ = wgc[...], wbi[...]
    x = (jnp.dot(slo_, wgc_[:HD], preferred_element_type=jnp.float32)
         + jnp.dot(shi_, wgc_[HD:], preferred_element_type=jnp.float32)
         + bgc[...])
    x = jnp.maximum(x, 0.2 * x)
    y = (jnp.dot(elo_ * slo_, wbi_[:HD], preferred_element_type=jnp.float32)
         + jnp.dot(ehi_ * shi_, wbi_[HD:], preferred_element_type=jnp.float32)
         + bbi[...])
    y = jnp.maximum(y, 0.2 * y)
    e = x + y
    n = jnp.sqrt(jnp.sum(e * e, axis=1, keepdims=True))
    nrm[...] = e / jnp.maximum(n, 1e-12)
    egon[0] = e[:, :HD]
    egon[1] = e[:, HD:]


_dense = pl.pallas_call(
    _dense_body,
    grid=(NBLK,),
    in_specs=[
        pl.BlockSpec((1, BN, HD), lambda i: (0, i, 0)),
        pl.BlockSpec((1, BN, HD), lambda i: (1, i, 0)),
        pl.BlockSpec((1, BN, HD), lambda i: (0, i, 0)),
        pl.BlockSpec((1, BN, HD), lambda i: (1, i, 0)),
        pl.BlockSpec((D, D), lambda i: (0, 0)),
        pl.BlockSpec((1, D), lambda i: (0, 0)),
        pl.BlockSpec((D, D), lambda i: (0, 0)),
        pl.BlockSpec((1, D), lambda i: (0, 0)),
    ],
    out_specs=[
        pl.BlockSpec((2, BN, HD), lambda i: (0, i, 0)),
        pl.BlockSpec((BN, D), lambda i: (i, 0)),
    ],
    out_shape=[
        jax.ShapeDtypeStruct((2, NP, HD), jnp.float32),
        jax.ShapeDtypeStruct((NP, D), jnp.float32),
    ],
)


GB = BATCH // (NC * NS)


@functools.partial(
    pl.kernel,
    out_type=(jax.ShapeDtypeStruct((4, BATCH, D), jnp.float32),
              jax.ShapeDtypeStruct((4, BATCH, D), jnp.float32)),
    mesh=_SC_MESH,
    scratch_types=[
        pltpu.VMEM((GB,), jnp.int32),
        pltpu.VMEM((GB, D), jnp.float32),
        pltpu.SemaphoreType.DMA,
    ],
    compiler_params=pltpu.CompilerParams(use_tc_tiling_on_sc=False),
)
def _gather(t0, t1, t2, t3, uidx_h, iidx_h, uout, iout, idxv, rows, sem):
    c = lax.axis_index("c")
    s = lax.axis_index("s")
    wid = s * NC + c
    base = wid * GB
    tables = (t0, t1, t2, t3)
    pltpu.sync_copy(uidx_h.at[pl.ds(base, GB)], idxv)
    for t in range(4):
        pltpu.async_copy(tables[t].at[idxv], rows, sem).wait()
        pltpu.sync_copy(rows, uout.at[t, pl.ds(base, GB)])
    pltpu.sync_copy(iidx_h.at[pl.ds(base, GB)], idxv)
    for t in range(4):
        pltpu.async_copy(tables[t].at[idxv], rows, sem).wait()
        pltpu.sync_copy(rows, iout.at[t, pl.ds(base, GB)])


DB = 512


def _dot_body(u, v, o):
    o[...] = jnp.sum(u[...] * v[...], axis=(0, 2)).reshape(1, 1, DB)


_dot = pl.pallas_call(
    _dot_body,
    grid=(BATCH // DB,),
    in_specs=[
        pl.BlockSpec((4, DB, D), lambda i: (0, i, 0)),
        pl.BlockSpec((4, DB, D), lambda i: (0, i, 0)),
    ],
    out_specs=pl.BlockSpec((1, 1, DB), lambda i: (i, 0, 0)),
    out_shape=jax.ShapeDtypeStruct((BATCH // DB, 1, DB), jnp.float32),
)


def kernel(users, items, adj_rows, adj_cols, adj_vals, user_emb, item_emb,
           W_gc_0, b_gc_0, W_bi_0, b_bi_0,
           W_gc_1, b_gc_1, W_bi_1, b_bi_1,
           W_gc_2, b_gc_2, W_bi_2, b_bi_2):
    ego0 = jnp.concatenate([user_emb, item_emb], axis=0)
    zrow = jnp.zeros((NP - NN, HD), jnp.float32)
    ego2 = jnp.concatenate([user_emb[:, :HD], item_emb[:, :HD], zrow,
                            user_emb[:, HD:], item_emb[:, HD:], zrow], axis=0)
    pad = NNZ_PAD - NNZ
    colp = jnp.concatenate([adj_cols.astype(jnp.int32),
                            jnp.zeros((pad,), jnp.int32)])
    rowp = jnp.concatenate([adj_rows.astype(jnp.int32),
                            jnp.zeros((pad,), jnp.int32)])
    valp = jnp.concatenate([adj_vals, jnp.zeros((pad,), jnp.float32)])
    col2 = jnp.stack([colp, colp + NP]).reshape(2, NS, SUPS, CPS, CH)
    rowr = rowp.reshape(NS, SUPS, CPS, CH)
    valr = jnp.broadcast_to(valp[:, None],
                            (NNZ_PAD, 16)).reshape(NS, SUPS, CPS * CH, 16)
    Ws = [(W_gc_0, b_gc_0, W_bi_0, b_bi_0),
          (W_gc_1, b_gc_1, W_bi_1, b_bi_1),
          (W_gc_2, b_gc_2, W_bi_2, b_bi_2)]
    zero_blk = jnp.zeros((ZROWS, HD), jnp.float32)
    norms = []
    for k in range(3):
        side2 = _spmm(ego2, col2, valr, rowr, zero_blk)
        egon, nrm = _dense(side2.reshape(2, NP, HD), side2.reshape(2, NP, HD),
                           ego2.reshape(2, NP, HD), ego2.reshape(2, NP, HD),
                           *Ws[k])
        ego2 = egon.reshape(2 * NP, HD)
        norms.append(nrm)
    uidx = users.astype(jnp.int32)
    iidx = items.astype(jnp.int32) + NUSERS
    ug, ig = _gather(ego0, norms[0], norms[1], norms[2], uidx, iidx)
    return _dot(ug, ig).reshape(BATCH)


# speedup vs baseline: 5.1426x; 1.2795x over previous
import functools

import jax
import jax.numpy as jnp
from jax import lax
from jax.experimental import pallas as pl
from jax.experimental.pallas import tpu as pltpu
from jax.experimental.pallas import tpu_sc as plsc

NUSERS = 25000
NN = 50000
NP = 51200
D = 64
HD = 32
NNZ = 800000
BATCH = 4096
NC, NS = 2, 16

CH = 128
CPS = 4
SUPS = 98
EDGES_SUB = SUPS * CPS * CH
NNZ_PAD = EDGES_SUB * NS

RPS = NP // NS
ZROWS = 640
ZK = RPS // ZROWS

_SC_MESH = plsc.VectorSubcoreMesh(core_axis_name="c", subcore_axis_name="s")


@functools.partial(
    pl.kernel,
    out_type=jax.ShapeDtypeStruct((2 * NP, HD), jnp.float32),
    mesh=_SC_MESH,
    scratch_types=[
        pltpu.VMEM_SHARED((NP, HD), jnp.float32),
        pltpu.VMEM((CPS, CH), jnp.int32),
        pltpu.VMEM((CPS * CH,), jnp.float32),
        pltpu.VMEM((CPS, CH), jnp.int32),
        pltpu.VMEM((CPS * CH, HD), jnp.float32),
        pltpu.SemaphoreType.DMA,
    ],
    compiler_params=pltpu.CompilerParams(use_tc_tiling_on_sc=False),
)
def _spmm(ego_h, col_h, val_h, row_h, zero_h, out_h,
          side, colblk, valblk, rowblk, gath, sem):
    c = lax.axis_index("c")
    s = lax.axis_index("s")

    for k in range(ZK):
        pltpu.sync_copy(zero_h, side.at[pl.ds(s * RPS + k * ZROWS, ZROWS)])
    plsc.subcore_barrier()

    def _sup(u, _):
        pltpu.sync_copy(col_h.at[c, s, u], colblk)
        pltpu.sync_copy(val_h.at[s, u], valblk)
        pltpu.sync_copy(row_h.at[s, u], rowblk)
        cps = [
            pltpu.async_copy(ego_h.at[colblk.at[j]],
                             gath.at[pl.ds(j * CH, CH)], sem)
            for j in range(CPS)
        ]
        for cp in cps:
            cp.wait()
        @plsc.parallel_loop(0, CPS * CH // 16, unroll=2)
        def _scale(g):
            e0 = g * 16
            vals = valblk[pl.ds(e0, 16)]
            for l in range(16):
                v = jnp.full((16,), vals[l])
                a = gath[e0 + l, pl.ds(0, 16)]
                b = gath[e0 + l, pl.ds(16, 16)]
                gath[e0 + l, pl.ds(0, 16)] = a * v
                gath[e0 + l, pl.ds(16, 16)] = b * v
        for j in range(CPS):
            pltpu.sync_copy(gath.at[pl.ds(j * CH, CH)],
                            side.at[rowblk.at[j]], add=True)
        return 0
    lax.fori_loop(0, SUPS, _sup, 0)
    plsc.subcore_barrier()

    for k in range(ZK):
        start = s * RPS + k * ZROWS
        pltpu.sync_copy(side.at[pl.ds(start, ZROWS)],
                        out_h.at[pl.ds(c * NP + start, ZROWS)])


BN = 1600
NBLK = NP // BN


def _dense_body(slo, shi, elo, ehi, wgc, bgc, wbi, bbi, egon, nrm):
    slo_, shi_ = slo[0], shi[0]
    elo_, ehi_ = elo[0], ehi[0]
    wgc_, wbi_ = wgc[...], wbi[...]
    x = (jnp.dot(slo_, wgc_[:HD], preferred_element_type=jnp.float32)
         + jnp.dot(shi_, wgc_[HD:], preferred_element_type=jnp.float32)
         + bgc[...])
    x = jnp.maximum(x, 0.2 * x)
    y = (jnp.dot(elo_ * slo_, wbi_[:HD], preferred_element_type=jnp.float32)
         + jnp.dot(ehi_ * shi_, wbi_[HD:], preferred_element_type=jnp.float32)
         + bbi[...])
    y = jnp.maximum(y, 0.2 * y)
    e = x + y
    n = jnp.sqrt(jnp.sum(e * e, axis=1, keepdims=True))
    nrm[...] = e / jnp.maximum(n, 1e-12)
    egon[0] = e[:, :HD]
    egon[1] = e[:, HD:]


_dense = pl.pallas_call(
    _dense_body,
    grid=(NBLK,),
    in_specs=[
        pl.BlockSpec((1, BN, HD), lambda i: (0, i, 0)),
        pl.BlockSpec((1, BN, HD), lambda i: (1, i, 0)),
        pl.BlockSpec((1, BN, HD), lambda i: (0, i, 0)),
        pl.BlockSpec((1, BN, HD), lambda i: (1, i, 0)),
        pl.BlockSpec((D, D), lambda i: (0, 0)),
        pl.BlockSpec((1, D), lambda i: (0, 0)),
        pl.BlockSpec((D, D), lambda i: (0, 0)),
        pl.BlockSpec((1, D), lambda i: (0, 0)),
    ],
    out_specs=[
        pl.BlockSpec((2, BN, HD), lambda i: (0, i, 0)),
        pl.BlockSpec((BN, D), lambda i: (i, 0)),
    ],
    out_shape=[
        jax.ShapeDtypeStruct((2, NP, HD), jnp.float32),
        jax.ShapeDtypeStruct((NP, D), jnp.float32),
    ],
)


GB = BATCH // (NC * NS)


@functools.partial(
    pl.kernel,
    out_type=(jax.ShapeDtypeStruct((4, BATCH, D), jnp.float32),
              jax.ShapeDtypeStruct((4, BATCH, D), jnp.float32)),
    mesh=_SC_MESH,
    scratch_types=[
        pltpu.VMEM((GB,), jnp.int32),
        pltpu.VMEM((GB, D), jnp.float32),
        pltpu.SemaphoreType.DMA,
    ],
    compiler_params=pltpu.CompilerParams(use_tc_tiling_on_sc=False),
)
def _gather(t0, t1, t2, t3, uidx_h, iidx_h, uout, iout, idxv, rows, sem):
    c = lax.axis_index("c")
    s = lax.axis_index("s")
    wid = s * NC + c
    base = wid * GB
    tables = (t0, t1, t2, t3)
    pltpu.sync_copy(uidx_h.at[pl.ds(base, GB)], idxv)
    for t in range(4):
        pltpu.async_copy(tables[t].at[idxv], rows, sem).wait()
        pltpu.sync_copy(rows, uout.at[t, pl.ds(base, GB)])
    pltpu.sync_copy(iidx_h.at[pl.ds(base, GB)], idxv)
    for t in range(4):
        pltpu.async_copy(tables[t].at[idxv], rows, sem).wait()
        pltpu.sync_copy(rows, iout.at[t, pl.ds(base, GB)])


DB = 512


def _dot_body(u, v, o):
    o[...] = jnp.sum(u[...] * v[...], axis=(0, 2)).reshape(1, 1, DB)


_dot = pl.pallas_call(
    _dot_body,
    grid=(BATCH // DB,),
    in_specs=[
        pl.BlockSpec((4, DB, D), lambda i: (0, i, 0)),
        pl.BlockSpec((4, DB, D), lambda i: (0, i, 0)),
    ],
    out_specs=pl.BlockSpec((1, 1, DB), lambda i: (i, 0, 0)),
    out_shape=jax.ShapeDtypeStruct((BATCH // DB, 1, DB), jnp.float32),
)


def kernel(users, items, adj_rows, adj_cols, adj_vals, user_emb, item_emb,
           W_gc_0, b_gc_0, W_bi_0, b_bi_0,
           W_gc_1, b_gc_1, W_bi_1, b_bi_1,
           W_gc_2, b_gc_2, W_bi_2, b_bi_2):
    ego0 = jnp.concatenate([user_emb, item_emb], axis=0)
    zrow = jnp.zeros((NP - NN, HD), jnp.float32)
    ego2 = jnp.concatenate([user_emb[:, :HD], item_emb[:, :HD], zrow,
                            user_emb[:, HD:], item_emb[:, HD:], zrow], axis=0)
    pad = NNZ_PAD - NNZ
    colp = jnp.concatenate([adj_cols.astype(jnp.int32),
                            jnp.zeros((pad,), jnp.int32)])
    rowp = jnp.concatenate([adj_rows.astype(jnp.int32),
                            jnp.zeros((pad,), jnp.int32)])
    valp = jnp.concatenate([adj_vals, jnp.zeros((pad,), jnp.float32)])
    col2 = jnp.stack([colp, colp + NP]).reshape(2, NS, SUPS, CPS, CH)
    rowr = rowp.reshape(NS, SUPS, CPS, CH)
    valr = valp.reshape(NS, SUPS, CPS * CH)
    Ws = [(W_gc_0, b_gc_0, W_bi_0, b_bi_0),
          (W_gc_1, b_gc_1, W_bi_1, b_bi_1),
          (W_gc_2, b_gc_2, W_bi_2, b_bi_2)]
    zero_blk = jnp.zeros((ZROWS, HD), jnp.float32)
    norms = []
    for k in range(3):
        side2 = _spmm(ego2, col2, valr, rowr, zero_blk)
        egon, nrm = _dense(side2.reshape(2, NP, HD), side2.reshape(2, NP, HD),
                           ego2.reshape(2, NP, HD), ego2.reshape(2, NP, HD),
                           *Ws[k])
        ego2 = egon.reshape(2 * NP, HD)
        norms.append(nrm)
    uidx = users.astype(jnp.int32)
    iidx = items.astype(jnp.int32) + NUSERS
    ug, ig = _gather(ego0, norms[0], norms[1], norms[2], uidx, iidx)
    return _dot(ug, ig).reshape(BATCH)


# speedup vs baseline: 6.4453x; 1.2533x over previous
import functools

import jax
import jax.numpy as jnp
from jax import lax
from jax.experimental import pallas as pl
from jax.experimental.pallas import tpu as pltpu
from jax.experimental.pallas import tpu_sc as plsc

NUSERS = 25000
NN = 50000
NP = 51200
D = 64
HD = 32
NNZ = 800000
BATCH = 4096
NC, NS = 2, 16

CH = 128
CPS = 2
SUPS = 196
EDGES_SUB = SUPS * CPS * CH
NNZ_PAD = EDGES_SUB * NS
IDXR = 2 * CPS
NSLOT = 4

RPS = NP // NS
ZROWS = 640
ZK = RPS // ZROWS

_SC_MESH = plsc.VectorSubcoreMesh(core_axis_name="c", subcore_axis_name="s")


@functools.partial(
    pl.kernel,
    out_type=jax.ShapeDtypeStruct((2 * NP, HD), jnp.float32),
    mesh=_SC_MESH,
    scratch_types=[
        pltpu.VMEM_SHARED((NP, HD), jnp.float32),
        pltpu.VMEM((NSLOT * IDXR, CH), jnp.int32),
        pltpu.VMEM((NSLOT, CPS * CH), jnp.float32),
        pltpu.VMEM((2, CPS * CH, HD), jnp.float32),
        pltpu.SemaphoreType.DMA,
        pltpu.SemaphoreType.DMA,
        pltpu.SemaphoreType.DMA,
        pltpu.SemaphoreType.DMA,
        pltpu.SemaphoreType.DMA,
        pltpu.SemaphoreType.DMA,
        pltpu.SemaphoreType.DMA,
        pltpu.SemaphoreType.DMA,
        pltpu.SemaphoreType.DMA,
    ],
    compiler_params=pltpu.CompilerParams(use_tc_tiling_on_sc=False),
)
def _spmm(ego_h, cvr_h, val_h, zero_h, out_h, side, idxb, valb, gath,
          semg, si0, si1, si2, si3, ss0, ss1, ss2, ss3):
    semi = (si0, si1, si2, si3)
    sesc = (ss0, ss1, ss2, ss3)
    c = lax.axis_index("c")
    s = lax.axis_index("s")

    for k in range(ZK):
        pltpu.sync_copy(zero_h, side.at[pl.ds(s * RPS + k * ZROWS, ZROWS)])
    plsc.subcore_barrier()

    def fire_idx(su, t):
        pltpu.async_copy(cvr_h.at[c, s, su], idxb.at[pl.ds(t * IDXR, IDXR)],
                         semi[t])
        pltpu.async_copy(val_h.at[s, su], valb.at[t], semi[t])

    def wait_idx(su, t):
        pltpu.make_async_copy(cvr_h.at[c, s, su],
                              idxb.at[pl.ds(t * IDXR, IDXR)], semi[t]).wait()
        pltpu.make_async_copy(val_h.at[s, su], valb.at[t], semi[t]).wait()

    def fire_gath(t, b):
        for j in range(CPS):
            pltpu.async_copy(ego_h.at[idxb.at[t * IDXR + j]],
                             gath.at[b, pl.ds(j * CH, CH)], semg)

    def wait_gath(t, b):
        for j in range(CPS):
            pltpu.make_async_copy(ego_h.at[idxb.at[t * IDXR + j]],
                                  gath.at[b, pl.ds(j * CH, CH)], semg).wait()

    def fire_scat(t, b):
        for j in range(CPS):
            pltpu.async_copy(gath.at[b, pl.ds(j * CH, CH)],
                             side.at[idxb.at[t * IDXR + CPS + j]],
                             sesc[t], add=True)

    def wait_scat(t, b):
        for j in range(CPS):
            pltpu.make_async_copy(gath.at[b, pl.ds(j * CH, CH)],
                                  side.at[idxb.at[t * IDXR + CPS + j]],
                                  sesc[t]).wait()

    def scale(t, b):
        for j in range(CPS):

            @plsc.parallel_loop(0, CH // 16, unroll=2)
            def _grp(g):
                e0 = j * CH + g * 16
                vals = valb[t, pl.ds(e0, 16)]
                for l in range(16):
                    v = jnp.full((16,), vals[l])
                    a = gath[b, e0 + l, pl.ds(0, 16)]
                    bb = gath[b, e0 + l, pl.ds(16, 16)]
                    gath[b, e0 + l, pl.ds(0, 16)] = a * v
                    gath[b, e0 + l, pl.ds(16, 16)] = bb * v

    fire_idx(0, 0)
    fire_idx(1, 1)
    wait_idx(0, 0)
    fire_gath(0, 0)

    def _iter(k, _):
        su0 = k * NSLOT
        for p in range(NSLOT):
            su = su0 + p
            t, b = p, p % 2
            t2 = (p + 2) % NSLOT
            wait_gath(t, b)
            scale(t, b)
            fire_scat(t, b)

            @pl.when(su >= 2)
            def _():
                wait_scat(t2, b)

            @pl.when(su + 2 <= SUPS - 1)
            def _():
                fire_idx(su + 2, t2)

            @pl.when(su + 1 <= SUPS - 1)
            def _():
                wait_idx(su + 1, (p + 1) % NSLOT)
                fire_gath((p + 1) % NSLOT, (p + 1) % 2)
        return 0
    lax.fori_loop(0, SUPS // NSLOT, _iter, 0)
    wait_scat(2, 0)
    wait_scat(3, 1)
    plsc.subcore_barrier()

    for k in range(ZK):
        start = s * RPS + k * ZROWS
        pltpu.sync_copy(side.at[pl.ds(start, ZROWS)],
                        out_h.at[pl.ds(c * NP + start, ZROWS)])


BN = 1600
NBLK = NP // BN


def _dense_body(slo, shi, elo, ehi, wgc, bgc, wbi, bbi, egon, nrm):
    slo_, shi_ = slo[0], shi[0]
    elo_, ehi_ = elo[0], ehi[0]
    wgc_, wbi_ = wgc[...], wbi[...]
    x = (jnp.dot(slo_, wgc_[:HD], preferred_element_type=jnp.float32)
         + jnp.dot(shi_, wgc_[HD:], preferred_element_type=jnp.float32)
         + bgc[...])
    x = jnp.maximum(x, 0.2 * x)
    y = (jnp.dot(elo_ * slo_, wbi_[:HD], preferred_element_type=jnp.float32)
         + jnp.dot(ehi_ * shi_, wbi_[HD:], preferred_element_type=jnp.float32)
         + bbi[...])
    y = jnp.maximum(y, 0.2 * y)
    e = x + y
    n = jnp.sqrt(jnp.sum(e * e, axis=1, keepdims=True))
    nrm[...] = e / jnp.maximum(n, 1e-12)
    egon[0] = e[:, :HD]
    egon[1] = e[:, HD:]


_dense = pl.pallas_call(
    _dense_body,
    grid=(NBLK,),
    in_specs=[
        pl.BlockSpec((1, BN, HD), lambda i: (0, i, 0)),
        pl.BlockSpec((1, BN, HD), lambda i: (1, i, 0)),
        pl.BlockSpec((1, BN, HD), lambda i: (0, i, 0)),
        pl.BlockSpec((1, BN, HD), lambda i: (1, i, 0)),
        pl.BlockSpec((D, D), lambda i: (0, 0)),
        pl.BlockSpec((1, D), lambda i: (0, 0)),
        pl.BlockSpec((D, D), lambda i: (0, 0)),
        pl.BlockSpec((1, D), lambda i: (0, 0)),
    ],
    out_specs=[
        pl.BlockSpec((2, BN, HD), lambda i: (0, i, 0)),
        pl.BlockSpec((BN, D), lambda i: (i, 0)),
    ],
    out_shape=[
        jax.ShapeDtypeStruct((2, NP, HD), jnp.float32),
        jax.ShapeDtypeStruct((NP, D), jnp.float32),
    ],
)


GB = BATCH // (NC * NS)


@functools.partial(
    pl.kernel,
    out_type=(jax.ShapeDtypeStruct((4, BATCH, D), jnp.float32),
              jax.ShapeDtypeStruct((4, BATCH, D), jnp.float32)),
    mesh=_SC_MESH,
    scratch_types=[
        pltpu.VMEM((GB,), jnp.int32),
        pltpu.VMEM((GB, D), jnp.float32),
        pltpu.SemaphoreType.DMA,
    ],
    compiler_params=pltpu.CompilerParams(use_tc_tiling_on_sc=False),
)
def _gather(t0, t1, t2, t3, uidx_h, iidx_h, uout, iout, idxv, rows, sem):
    c = lax.axis_index("c")
    s = lax.axis_index("s")
    wid = s * NC + c
    base = wid * GB
    tables = (t0, t1, t2, t3)
    pltpu.sync_copy(uidx_h.at[pl.ds(base, GB)], idxv)
    for t in range(4):
        pltpu.async_copy(tables[t].at[idxv], rows, sem).wait()
        pltpu.sync_copy(rows, uout.at[t, pl.ds(base, GB)])
    pltpu.sync_copy(iidx_h.at[pl.ds(base, GB)], idxv)
    for t in range(4):
        pltpu.async_copy(tables[t].at[idxv], rows, sem).wait()
        pltpu.sync_copy(rows, iout.at[t, pl.ds(base, GB)])


DB = 512


def _dot_body(u, v, o):
    o[...] = jnp.sum(u[...] * v[...], axis=(0, 2)).reshape(1, 1, DB)


_dot = pl.pallas_call(
    _dot_body,
    grid=(BATCH // DB,),
    in_specs=[
        pl.BlockSpec((4, DB, D), lambda i: (0, i, 0)),
        pl.BlockSpec((4, DB, D), lambda i: (0, i, 0)),
    ],
    out_specs=pl.BlockSpec((1, 1, DB), lambda i: (i, 0, 0)),
    out_shape=jax.ShapeDtypeStruct((BATCH // DB, 1, DB), jnp.float32),
)


def kernel(users, items, adj_rows, adj_cols, adj_vals, user_emb, item_emb,
           W_gc_0, b_gc_0, W_bi_0, b_bi_0,
           W_gc_1, b_gc_1, W_bi_1, b_bi_1,
           W_gc_2, b_gc_2, W_bi_2, b_bi_2):
    ego0 = jnp.concatenate([user_emb, item_emb], axis=0)
    zrow = jnp.zeros((NP - NN, HD), jnp.float32)
    ego2 = jnp.concatenate([user_emb[:, :HD], item_emb[:, :HD], zrow,
                            user_emb[:, HD:], item_emb[:, HD:], zrow], axis=0)
    pad = NNZ_PAD - NNZ
    colp = jnp.concatenate([adj_cols.astype(jnp.int32),
                            jnp.zeros((pad,), jnp.int32)])
    rowp = jnp.concatenate([adj_rows.astype(jnp.int32),
                            jnp.zeros((pad,), jnp.int32)])
    valp = jnp.concatenate([adj_vals, jnp.zeros((pad,), jnp.float32)])
    valr = valp.reshape(NS, SUPS, CPS * CH)

    def _mk(colc):
        x = jnp.stack([colc, rowp]).reshape(2, NS, SUPS, CPS, CH)
        return x.transpose(1, 2, 0, 3, 4).reshape(NS, SUPS, IDXR, CH)
    cvr = jnp.stack([_mk(colp), _mk(colp + NP)])
    Ws = [(W_gc_0, b_gc_0, W_bi_0, b_bi_0),
          (W_gc_1, b_gc_1, W_bi_1, b_bi_1),
          (W_gc_2, b_gc_2, W_bi_2, b_bi_2)]
    zero_blk = jnp.zeros((ZROWS, HD), jnp.float32)
    norms = []
    for k in range(3):
        side2 = _spmm(ego2, cvr, valr, zero_blk)
        egon, nrm = _dense(side2.reshape(2, NP, HD), side2.reshape(2, NP, HD),
                           ego2.reshape(2, NP, HD), ego2.reshape(2, NP, HD),
                           *Ws[k])
        ego2 = egon.reshape(2 * NP, HD)
        norms.append(nrm)
    uidx = users.astype(jnp.int32)
    iidx = items.astype(jnp.int32) + NUSERS
    ug, ig = _gather(ego0, norms[0], norms[1], norms[2], uidx, iidx)
    return _dot(ug, ig).reshape(BATCH)


# speedup vs baseline: 7.2027x; 1.1175x over previous
import functools

import jax
import jax.numpy as jnp
from jax import lax
from jax.experimental import pallas as pl
from jax.experimental.pallas import tpu as pltpu
from jax.experimental.pallas import tpu_sc as plsc

NUSERS = 25000
NN = 50000
NP = 51200
D = 64
HD = 32
NNZ = 800000
BATCH = 4096
NC, NS = 2, 16

CH = 128
CPS = 2
SUPS = 196
EDGES_SUB = SUPS * CPS * CH
NNZ_PAD = EDGES_SUB * NS
IDXR = 2 * CPS
NSLOT = 4

RPS = NP // NS
ZROWS = 640
ZK = RPS // ZROWS

_SC_MESH = plsc.VectorSubcoreMesh(core_axis_name="c", subcore_axis_name="s")


@functools.partial(
    pl.kernel,
    out_type=jax.ShapeDtypeStruct((2 * NP, HD), jnp.float32),
    mesh=_SC_MESH,
    scratch_types=[
        pltpu.VMEM_SHARED((NP, HD), jnp.float32),
        pltpu.VMEM((NSLOT * IDXR, CH), jnp.int32),
        pltpu.VMEM((NSLOT, CPS * CH), jnp.float32),
        pltpu.VMEM((2, CPS * CH, HD), jnp.float32),
        pltpu.SemaphoreType.DMA,
        pltpu.SemaphoreType.DMA,
        pltpu.SemaphoreType.DMA,
        pltpu.SemaphoreType.DMA,
        pltpu.SemaphoreType.DMA,
        pltpu.SemaphoreType.DMA,
        pltpu.SemaphoreType.DMA,
        pltpu.SemaphoreType.DMA,
        pltpu.SemaphoreType.DMA,
    ],
    compiler_params=pltpu.CompilerParams(use_tc_tiling_on_sc=False),
)
def _spmm(ego_h, cvr_h, val_h, zero_h, out_h, side, idxb, valb, gath,
          semg, si0, si1, si2, si3, ss0, ss1, ss2, ss3):
    semi = (si0, si1, si2, si3)
    sesc = (ss0, ss1, ss2, ss3)
    c = lax.axis_index("c")
    s = lax.axis_index("s")

    for k in range(ZK):
        pltpu.sync_copy(zero_h, side.at[pl.ds(s * RPS + k * ZROWS, ZROWS)])
    plsc.subcore_barrier()

    def fire_idx(su, t):
        pltpu.async_copy(cvr_h.at[c, s, su], idxb.at[pl.ds(t * IDXR, IDXR)],
                         semi[t])
        pltpu.async_copy(val_h.at[s, su], valb.at[t], semi[t])

    def wait_idx(su, t):
        pltpu.make_async_copy(cvr_h.at[c, s, su],
                              idxb.at[pl.ds(t * IDXR, IDXR)], semi[t]).wait()
        pltpu.make_async_copy(val_h.at[s, su], valb.at[t], semi[t]).wait()

    def fire_gath(t, b):
        for j in range(CPS):
            pltpu.async_copy(ego_h.at[idxb.at[t * IDXR + j]],
                             gath.at[b, pl.ds(j * CH, CH)], semg)

    def wait_gath(t, b):
        for j in range(CPS):
            pltpu.make_async_copy(ego_h.at[idxb.at[t * IDXR + j]],
                                  gath.at[b, pl.ds(j * CH, CH)], semg).wait()

    def fire_scat(t, b):
        for j in range(CPS):
            pltpu.async_copy(gath.at[b, pl.ds(j * CH, CH)],
                             side.at[idxb.at[t * IDXR + CPS + j]],
                             sesc[t], add=True)

    def wait_scat(t, b):
        for j in range(CPS):
            pltpu.make_async_copy(gath.at[b, pl.ds(j * CH, CH)],
                                  side.at[idxb.at[t * IDXR + CPS + j]],
                                  sesc[t]).wait()

    def scale(t, b):
        for j in range(CPS):

            @plsc.parallel_loop(0, CH // 16, unroll=2)
            def _grp(g):
                e0 = j * CH + g * 16
                vals = valb[t, pl.ds(e0, 16)]
                for l in range(16):
                    v = jnp.full((16,), vals[l])
                    a = gath[b, e0 + l, pl.ds(0, 16)]
                    bb = gath[b, e0 + l, pl.ds(16, 16)]
                    gath[b, e0 + l, pl.ds(0, 16)] = a * v
                    gath[b, e0 + l, pl.ds(16, 16)] = bb * v

    fire_idx(0, 0)
    fire_idx(1, 1)
    wait_idx(0, 0)
    fire_gath(0, 0)

    def _iter(k, _):
        su0 = k * NSLOT
        for p in range(NSLOT):
            su = su0 + p
            t, b = p, p % 2
            t2 = (p + 2) % NSLOT
            tn, bn = (p + 1) % NSLOT, (p + 1) % 2
            wait_gath(t, b)

            @pl.when(su >= 1)
            def _():
                wait_scat((p + 3) % NSLOT, bn)

            @pl.when(su + 1 <= SUPS - 1)
            def _():
                wait_idx(su + 1, tn)
                fire_gath(tn, bn)

            scale(t, b)
            fire_scat(t, b)

            @pl.when(su + 2 <= SUPS - 1)
            def _():
                fire_idx(su + 2, t2)
        return 0
    lax.fori_loop(0, SUPS // NSLOT, _iter, 0)
    wait_scat(3, 1)
    plsc.subcore_barrier()

    for k in range(ZK):
        start = s * RPS + k * ZROWS
        pltpu.sync_copy(side.at[pl.ds(start, ZROWS)],
                        out_h.at[pl.ds(c * NP + start, ZROWS)])


BN = 1600
NBLK = NP // BN


def _dense_body(slo, shi, elo, ehi, wgc, bgc, wbi, bbi, egon, nrm):
    slo_, shi_ = slo[0], shi[0]
    elo_, ehi_ = elo[0], ehi[0]
    wgc_, wbi_ = wgc[...], wbi[...]
    x = (jnp.dot(slo_, wgc_[:HD], preferred_element_type=jnp.float32)
         + jnp.dot(shi_, wgc_[HD:], preferred_element_type=jnp.float32)
         + bgc[...])
    x = jnp.maximum(x, 0.2 * x)
    y = (jnp.dot(elo_ * slo_, wbi_[:HD], preferred_element_type=jnp.float32)
         + jnp.dot(ehi_ * shi_, wbi_[HD:], preferred_element_type=jnp.float32)
         + bbi[...])
    y = jnp.maximum(y, 0.2 * y)
    e = x + y
    n = jnp.sqrt(jnp.sum(e * e, axis=1, keepdims=True))
    nrm[...] = e / jnp.maximum(n, 1e-12)
    egon[0] = e[:, :HD]
    egon[1] = e[:, HD:]


_dense = pl.pallas_call(
    _dense_body,
    grid=(NBLK,),
    in_specs=[
        pl.BlockSpec((1, BN, HD), lambda i: (0, i, 0)),
        pl.BlockSpec((1, BN, HD), lambda i: (1, i, 0)),
        pl.BlockSpec((1, BN, HD), lambda i: (0, i, 0)),
        pl.BlockSpec((1, BN, HD), lambda i: (1, i, 0)),
        pl.BlockSpec((D, D), lambda i: (0, 0)),
        pl.BlockSpec((1, D), lambda i: (0, 0)),
        pl.BlockSpec((D, D), lambda i: (0, 0)),
        pl.BlockSpec((1, D), lambda i: (0, 0)),
    ],
    out_specs=[
        pl.BlockSpec((2, BN, HD), lambda i: (0, i, 0)),
        pl.BlockSpec((BN, D), lambda i: (i, 0)),
    ],
    out_shape=[
        jax.ShapeDtypeStruct((2, NP, HD), jnp.float32),
        jax.ShapeDtypeStruct((NP, D), jnp.float32),
    ],
)


GB = BATCH // (NC * NS)


@functools.partial(
    pl.kernel,
    out_type=(jax.ShapeDtypeStruct((4, BATCH, D), jnp.float32),
              jax.ShapeDtypeStruct((4, BATCH, D), jnp.float32)),
    mesh=_SC_MESH,
    scratch_types=[
        pltpu.VMEM((GB,), jnp.int32),
        pltpu.VMEM((GB, D), jnp.float32),
        pltpu.SemaphoreType.DMA,
    ],
    compiler_params=pltpu.CompilerParams(use_tc_tiling_on_sc=False),
)
def _gather(t0, t1, t2, t3, uidx_h, iidx_h, uout, iout, idxv, rows, sem):
    c = lax.axis_index("c")
    s = lax.axis_index("s")
    wid = s * NC + c
    base = wid * GB
    tables = (t0, t1, t2, t3)
    pltpu.sync_copy(uidx_h.at[pl.ds(base, GB)], idxv)
    for t in range(4):
        pltpu.async_copy(tables[t].at[idxv], rows, sem).wait()
        pltpu.sync_copy(rows, uout.at[t, pl.ds(base, GB)])
    pltpu.sync_copy(iidx_h.at[pl.ds(base, GB)], idxv)
    for t in range(4):
        pltpu.async_copy(tables[t].at[idxv], rows, sem).wait()
        pltpu.sync_copy(rows, iout.at[t, pl.ds(base, GB)])


DB = 512


def _dot_body(u, v, o):
    o[...] = jnp.sum(u[...] * v[...], axis=(0, 2)).reshape(1, 1, DB)


_dot = pl.pallas_call(
    _dot_body,
    grid=(BATCH // DB,),
    in_specs=[
        pl.BlockSpec((4, DB, D), lambda i: (0, i, 0)),
        pl.BlockSpec((4, DB, D), lambda i: (0, i, 0)),
    ],
    out_specs=pl.BlockSpec((1, 1, DB), lambda i: (i, 0, 0)),
    out_shape=jax.ShapeDtypeStruct((BATCH // DB, 1, DB), jnp.float32),
)


def kernel(users, items, adj_rows, adj_cols, adj_vals, user_emb, item_emb,
           W_gc_0, b_gc_0, W_bi_0, b_bi_0,
           W_gc_1, b_gc_1, W_bi_1, b_bi_1,
           W_gc_2, b_gc_2, W_bi_2, b_bi_2):
    ego0 = jnp.concatenate([user_emb, item_emb], axis=0)
    zrow = jnp.zeros((NP - NN, HD), jnp.float32)
    ego2 = jnp.concatenate([user_emb[:, :HD], item_emb[:, :HD], zrow,
                            user_emb[:, HD:], item_emb[:, HD:], zrow], axis=0)
    pad = NNZ_PAD - NNZ
    colp = jnp.concatenate([adj_cols.astype(jnp.int32),
                            jnp.zeros((pad,), jnp.int32)])
    rowp = jnp.concatenate([adj_rows.astype(jnp.int32),
                            jnp.zeros((pad,), jnp.int32)])
    valp = jnp.concatenate([adj_vals, jnp.zeros((pad,), jnp.float32)])
    valr = valp.reshape(NS, SUPS, CPS * CH)

    def _mk(colc):
        x = jnp.stack([colc, rowp]).reshape(2, NS, SUPS, CPS, CH)
        return x.transpose(1, 2, 0, 3, 4).reshape(NS, SUPS, IDXR, CH)
    cvr = jnp.stack([_mk(colp), _mk(colp + NP)])
    Ws = [(W_gc_0, b_gc_0, W_bi_0, b_bi_0),
          (W_gc_1, b_gc_1, W_bi_1, b_bi_1),
          (W_gc_2, b_gc_2, W_bi_2, b_bi_2)]
    zero_blk = jnp.zeros((ZROWS, HD), jnp.float32)
    norms = []
    for k in range(3):
        side2 = _spmm(ego2, cvr, valr, zero_blk)
        egon, nrm = _dense(side2.reshape(2, NP, HD), side2.reshape(2, NP, HD),
                           ego2.reshape(2, NP, HD), ego2.reshape(2, NP, HD),
                           *Ws[k])
        ego2 = egon.reshape(2 * NP, HD)
        norms.append(nrm)
    uidx = users.astype(jnp.int32)
    iidx = items.astype(jnp.int32) + NUSERS
    ug, ig = _gather(ego0, norms[0], norms[1], norms[2], uidx, iidx)
    return _dot(ug, ig).reshape(BATCH)


# speedup vs baseline: 7.2640x; 1.0085x over previous
import functools

import jax
import jax.numpy as jnp
from jax import lax
from jax.experimental import pallas as pl
from jax.experimental.pallas import tpu as pltpu
from jax.experimental.pallas import tpu_sc as plsc

NUSERS = 25000
NN = 50000
NP = 51200
D = 64
HD = 32
NNZ = 800000
BATCH = 4096
NC, NS = 2, 16

CH = 128
CPS = 2
SUPS = 196
EDGES_SUB = SUPS * CPS * CH
NNZ_PAD = EDGES_SUB * NS
IDXR = 2 * CPS
NSLOT = 4

RPS = NP // NS
ZROWS = 640
ZK = RPS // ZROWS

_SC_MESH = plsc.VectorSubcoreMesh(core_axis_name="c", subcore_axis_name="s")


@functools.partial(
    pl.kernel,
    out_type=jax.ShapeDtypeStruct((2 * NP, HD), jnp.float32),
    mesh=_SC_MESH,
    scratch_types=[
        pltpu.VMEM_SHARED((NP, HD), jnp.float32),
        pltpu.VMEM((NSLOT * IDXR, CH), jnp.int32),
        pltpu.VMEM((NSLOT, CPS * CH), jnp.float32),
        pltpu.VMEM((2, CPS * CH, HD), jnp.float32),
        pltpu.SemaphoreType.DMA,
        pltpu.SemaphoreType.DMA,
        pltpu.SemaphoreType.DMA,
        pltpu.SemaphoreType.DMA,
        pltpu.SemaphoreType.DMA,
        pltpu.SemaphoreType.DMA,
        pltpu.SemaphoreType.DMA,
        pltpu.SemaphoreType.DMA,
        pltpu.SemaphoreType.DMA,
    ],
    compiler_params=pltpu.CompilerParams(use_tc_tiling_on_sc=False),
)
def _spmm(ego_h, col_h, row_h, val_h, zero_h, out_h, side, idxb, valb, gath,
          semg, si0, si1, si2, si3, ss0, ss1, ss2, ss3):
    semi = (si0, si1, si2, si3)
    sesc = (ss0, ss1, ss2, ss3)
    c = lax.axis_index("c")
    s = lax.axis_index("s")

    for k in range(ZK):
        pltpu.sync_copy(zero_h, side.at[pl.ds(s * RPS + k * ZROWS, ZROWS)])
    plsc.subcore_barrier()

    def fire_idx(su, t):
        pltpu.async_copy(col_h.at[c, s, su], idxb.at[pl.ds(t * IDXR, CPS)],
                         semi[t])
        pltpu.async_copy(row_h.at[s, su],
                         idxb.at[pl.ds(t * IDXR + CPS, CPS)], semi[t])
        pltpu.async_copy(val_h.at[s, su], valb.at[t], semi[t])

    def wait_idx(su, t):
        pltpu.make_async_copy(col_h.at[c, s, su],
                              idxb.at[pl.ds(t * IDXR, CPS)], semi[t]).wait()
        pltpu.make_async_copy(row_h.at[s, su],
                              idxb.at[pl.ds(t * IDXR + CPS, CPS)],
                              semi[t]).wait()
        pltpu.make_async_copy(val_h.at[s, su], valb.at[t], semi[t]).wait()

    def fire_gath(t, b):
        for j in range(CPS):
            pltpu.async_copy(ego_h.at[idxb.at[t * IDXR + j]],
                             gath.at[b, pl.ds(j * CH, CH)], semg)

    def wait_gath(t, b):
        for j in range(CPS):
            pltpu.make_async_copy(ego_h.at[idxb.at[t * IDXR + j]],
                                  gath.at[b, pl.ds(j * CH, CH)], semg).wait()

    def fire_scat(t, b):
        for j in range(CPS):
            pltpu.async_copy(gath.at[b, pl.ds(j * CH, CH)],
                             side.at[idxb.at[t * IDXR + CPS + j]],
                             sesc[t], add=True)

    def wait_scat(t, b):
        for j in range(CPS):
            pltpu.make_async_copy(gath.at[b, pl.ds(j * CH, CH)],
                                  side.at[idxb.at[t * IDXR + CPS + j]],
                                  sesc[t]).wait()

    def scale(t, b):
        for j in range(CPS):

            @plsc.parallel_loop(0, CH // 16, unroll=2)
            def _grp(g):
                e0 = j * CH + g * 16
                vals = valb[t, pl.ds(e0, 16)]
                for l in range(16):
                    v = jnp.full((16,), vals[l])
                    a = gath[b, e0 + l, pl.ds(0, 16)]
                    bb = gath[b, e0 + l, pl.ds(16, 16)]
                    gath[b, e0 + l, pl.ds(0, 16)] = a * v
                    gath[b, e0 + l, pl.ds(16, 16)] = bb * v

    fire_idx(0, 0)
    fire_idx(1, 1)
    wait_idx(0, 0)
    fire_gath(0, 0)

    def _iter(k, _):
        su0 = k * NSLOT
        for p in range(NSLOT):
            su = su0 + p
            t, b = p, p % 2
            t2 = (p + 2) % NSLOT
            tn, bn = (p + 1) % NSLOT, (p + 1) % 2
            wait_gath(t, b)

            @pl.when(su >= 1)
            def _():
                wait_scat((p + 3) % NSLOT, bn)

            @pl.when(su + 1 <= SUPS - 1)
            def _():
                wait_idx(su + 1, tn)
                fire_gath(tn, bn)

            scale(t, b)
            fire_scat(t, b)

            @pl.when(su + 2 <= SUPS - 1)
            def _():
                fire_idx(su + 2, t2)
        return 0
    lax.fori_loop(0, SUPS // NSLOT, _iter, 0)
    wait_scat(3, 1)
    plsc.subcore_barrier()

    for k in range(ZK):
        start = s * RPS + k * ZROWS
        pltpu.sync_copy(side.at[pl.ds(start, ZROWS)],
                        out_h.at[pl.ds(c * NP + start, ZROWS)])


BN = 3200
NBLK = NP // BN


def _dense_body(slo, shi, elo, ehi, wgc, bgc, wbi, bbi, egon, nrm):
    slo_, shi_ = slo[0], shi[0]
    elo_, ehi_ = elo[0], ehi[0]
    wgc_, wbi_ = wgc[...], wbi[...]
    x = (jnp.dot(slo_, wgc_[:HD], preferred_element_type=jnp.float32)
         + jnp.dot(shi_, wgc_[HD:], preferred_element_type=jnp.float32)
         + bgc[...])
    x = jnp.maximum(x, 0.2 * x)
    y = (jnp.dot(elo_ * slo_, wbi_[:HD], preferred_element_type=jnp.float32)
         + jnp.dot(ehi_ * shi_, wbi_[HD:], preferred_element_type=jnp.float32)
         + bbi[...])
    y = jnp.maximum(y, 0.2 * y)
    e = x + y
    n = jnp.sqrt(jnp.sum(e * e, axis=1, keepdims=True))
    nrm[...] = e / jnp.maximum(n, 1e-12)
    egon[0] = e[:, :HD]
    egon[1] = e[:, HD:]


_dense = pl.pallas_call(
    _dense_body,
    grid=(NBLK,),
    in_specs=[
        pl.BlockSpec((1, BN, HD), lambda i: (0, i, 0)),
        pl.BlockSpec((1, BN, HD), lambda i: (1, i, 0)),
        pl.BlockSpec((1, BN, HD), lambda i: (0, i, 0)),
        pl.BlockSpec((1, BN, HD), lambda i: (1, i, 0)),
        pl.BlockSpec((D, D), lambda i: (0, 0)),
        pl.BlockSpec((1, D), lambda i: (0, 0)),
        pl.BlockSpec((D, D), lambda i: (0, 0)),
        pl.BlockSpec((1, D), lambda i: (0, 0)),
    ],
    out_specs=[
        pl.BlockSpec((2, BN, HD), lambda i: (0, i, 0)),
        pl.BlockSpec((BN, D), lambda i: (i, 0)),
    ],
    out_shape=[
        jax.ShapeDtypeStruct((2, NP, HD), jnp.float32),
        jax.ShapeDtypeStruct((NP, D), jnp.float32),
    ],
)


GB = BATCH // (NC * NS)


@functools.partial(
    pl.kernel,
    out_type=(jax.ShapeDtypeStruct((4, BATCH, D), jnp.float32),
              jax.ShapeDtypeStruct((4, BATCH, D), jnp.float32)),
    mesh=_SC_MESH,
    scratch_types=[
        pltpu.VMEM((GB,), jnp.int32),
        pltpu.VMEM((GB, D), jnp.float32),
        pltpu.SemaphoreType.DMA,
    ],
    compiler_params=pltpu.CompilerParams(use_tc_tiling_on_sc=False),
)
def _gather(t0, t1, t2, t3, uidx_h, iidx_h, uout, iout, idxv, rows, sem):
    c = lax.axis_index("c")
    s = lax.axis_index("s")
    wid = s * NC + c
    base = wid * GB
    tables = (t0, t1, t2, t3)
    pltpu.sync_copy(uidx_h.at[pl.ds(base, GB)], idxv)
    for t in range(4):
        pltpu.async_copy(tables[t].at[idxv], rows, sem).wait()
        pltpu.sync_copy(rows, uout.at[t, pl.ds(base, GB)])
    pltpu.sync_copy(iidx_h.at[pl.ds(base, GB)], idxv)
    for t in range(4):
        pltpu.async_copy(tables[t].at[idxv], rows, sem).wait()
        pltpu.sync_copy(rows, iout.at[t, pl.ds(base, GB)])


DB = 1024


def _dot_body(u, v, o):
    o[...] = jnp.sum(u[...] * v[...], axis=(0, 2)).reshape(1, 1, DB)


_dot = pl.pallas_call(
    _dot_body,
    grid=(BATCH // DB,),
    in_specs=[
        pl.BlockSpec((4, DB, D), lambda i: (0, i, 0)),
        pl.BlockSpec((4, DB, D), lambda i: (0, i, 0)),
    ],
    out_specs=pl.BlockSpec((1, 1, DB), lambda i: (i, 0, 0)),
    out_shape=jax.ShapeDtypeStruct((BATCH // DB, 1, DB), jnp.float32),
)


def kernel(users, items, adj_rows, adj_cols, adj_vals, user_emb, item_emb,
           W_gc_0, b_gc_0, W_bi_0, b_bi_0,
           W_gc_1, b_gc_1, W_bi_1, b_bi_1,
           W_gc_2, b_gc_2, W_bi_2, b_bi_2):
    ego0 = jnp.concatenate([user_emb, item_emb], axis=0)
    zrow = jnp.zeros((NP - NN, HD), jnp.float32)
    ego2 = jnp.concatenate([user_emb[:, :HD], item_emb[:, :HD], zrow,
                            user_emb[:, HD:], item_emb[:, HD:], zrow], axis=0)
    pad = NNZ_PAD - NNZ
    colp = jnp.concatenate([adj_cols.astype(jnp.int32),
                            jnp.zeros((pad,), jnp.int32)])
    rowp = jnp.concatenate([adj_rows.astype(jnp.int32),
                            jnp.zeros((pad,), jnp.int32)])
    valp = jnp.concatenate([adj_vals, jnp.zeros((pad,), jnp.float32)])
    valr = valp.reshape(NS, SUPS, CPS * CH)
    col3 = jnp.stack([colp, colp + NP]).reshape(2, NS, SUPS, CPS, CH)
    row3 = rowp.reshape(NS, SUPS, CPS, CH)
    Ws = [(W_gc_0, b_gc_0, W_bi_0, b_bi_0),
          (W_gc_1, b_gc_1, W_bi_1, b_bi_1),
          (W_gc_2, b_gc_2, W_bi_2, b_bi_2)]
    zero_blk = jnp.zeros((ZROWS, HD), jnp.float32)
    norms = []
    for k in range(3):
        side2 = _spmm(ego2, col3, row3, valr, zero_blk)
        egon, nrm = _dense(side2.reshape(2, NP, HD), side2.reshape(2, NP, HD),
                           ego2.reshape(2, NP, HD), ego2.reshape(2, NP, HD),
                           *Ws[k])
        ego2 = egon.reshape(2 * NP, HD)
        norms.append(nrm)
    uidx = users.astype(jnp.int32)
    iidx = items.astype(jnp.int32) + NUSERS
    ug, ig = _gather(ego0, norms[0], norms[1], norms[2], uidx, iidx)
    return _dot(ug, ig).reshape(BATCH)
